# baseline (speedup 1.0000x reference)
import functools

import jax
import jax.numpy as jnp
from jax import lax
from jax.experimental import pallas as pl
from jax.experimental.pallas import tpu as pltpu
from jax.experimental.pallas import tpu_sc as plsc

_L = 16


@functools.cache
def _make_kernel(b, h, v, d):
    info = plsc.get_sparse_core_info()
    nc, ns = info.num_cores, info.num_subcores
    nw = nc * ns
    bw = b // nw
    mesh = plsc.VectorSubcoreMesh(core_axis_name="c", subcore_axis_name="s")

    @functools.partial(
        pl.kernel,
        mesh=mesh,
        out_type=jax.ShapeDtypeStruct((h, d, b), jnp.float32),
        compiler_params=pltpu.CompilerParams(
            use_tc_tiling_on_sc=False, needs_layout_passes=False
        ),
        scratch_types=[
            pltpu.VMEM((h, bw), jnp.int32),
            pltpu.VMEM((4, bw, d), jnp.float32),
            pltpu.VMEM((2, d, bw + 1), jnp.float32),
        ]
        + [pltpu.SemaphoreType.DMA] * 6,
    )
    def k(xt_hbm, table_hbm, out_hbm, xv, rows, tst, g0, g1, g2, g3, s0, s1):
        gsem = (g0, g1, g2, g3)
        ssem = (s0, s1)
        wid = lax.axis_index("s") * nc + lax.axis_index("c")
        b0 = wid * bw
        pltpu.sync_copy(xt_hbm.at[:, pl.ds(b0, bw)], xv)

        def fire_gather(hh, buf):
            pltpu.async_copy(table_hbm.at[xv.at[hh]], rows.at[buf], gsem[buf])

        def wait_gather(hh, buf):
            pltpu.make_async_copy(
                table_hbm.at[xv.at[hh]], rows.at[buf], gsem[buf]
            ).wait()

        def fire_store(hh, buf):
            pltpu.async_copy(
                tst.at[buf, :, pl.ds(0, bw)],
                out_hbm.at[hh, :, pl.ds(b0, bw)],
                ssem[buf],
            )

        def wait_store(hh, buf):
            pltpu.make_async_copy(
                tst.at[buf, :, pl.ds(0, bw)],
                out_hbm.at[hh, :, pl.ds(b0, bw)],
                ssem[buf],
            ).wait()

        cqs = [lax.iota(jnp.int32, _L) + _L * q for q in range(d // _L)]

        def transpose(gbuf, tbuf):
            src = rows.at[gbuf]
            dst = tst.at[tbuf]
            nq = d // _L

            def grp(i, carry):
                j0 = i * 32
                for j in range(0, 32, 4):
                    vals = [
                        [src[j0 + j + u, pl.ds(_L * q, _L)] for q in range(nq)]
                        for u in range(4)
                    ]
                    jvs = [
                        jnp.full((_L,), j + u, jnp.int32) + j0 for u in range(4)
                    ]
                    for u in range(4):
                        for q in range(nq):
                            plsc.store_scatter(dst, [cqs[q], jvs[u]], vals[u][q])
                return carry

            lax.fori_loop(0, bw // 32, grp, 0)

        def step(hh, u, wait_st, fire_g):
            wait_gather(hh, u)
            if wait_st:
                wait_store(hh - 2, u % 2)
            transpose(u, u % 2)
            fire_store(hh, u % 2)
            if fire_g:
                fire_gather(hh + 3, (u + 3) % 4)

        for hh in range(3):
            fire_gather(hh, hh)
        for hh in range(4):
            step(hh, hh, wait_st=hh >= 2, fire_g=True)

        def body(gidx, carry):
            h0 = 4 + gidx * 4
            for u in range(4):
                step(h0 + u, u, wait_st=True, fire_g=True)
            return carry

        lax.fori_loop(0, (h - 8) // 4, body, 0)
        for hh in range(h - 4, h):
            step(hh, hh % 4, wait_st=True, fire_g=hh + 3 < h)
        for u in range(2):
            wait_store(h - 2 + u, u)

    return k


@functools.cache
def _make_table_transpose(v, d):
    info = plsc.get_sparse_core_info()
    nc, ns = info.num_cores, info.num_subcores
    nw = nc * ns
    vb = 128
    nblk_total = -(-v // vb)
    per_w = -(-nblk_total // nw)
    per_w += per_w % 2
    last_blk = nblk_total - 1
    last_off = v - vb
    mesh = plsc.VectorSubcoreMesh(core_axis_name="c", subcore_axis_name="s")

    @functools.partial(
        pl.kernel,
        mesh=mesh,
        out_type=jax.ShapeDtypeStruct((v, d), jnp.float32),
        compiler_params=pltpu.CompilerParams(
            use_tc_tiling_on_sc=False, needs_layout_passes=False
        ),
        scratch_types=[
            pltpu.VMEM((2, d, vb), jnp.float32),
            pltpu.VMEM((2, vb, d + 1), jnp.float32),
        ]
        + [pltpu.SemaphoreType.DMA] * 4,
    )
    def k(tt_hbm, out_hbm, src, dst, i0, i1, o0, o1):
        isem = (i0, i1)
        osem = (o0, o1)
        wid = lax.axis_index("s") * nc + lax.axis_index("c")

        def voff(i):
            blk = jnp.minimum(wid * (per_w - 1) + i, last_blk)
            return jnp.minimum(blk * vb, last_off)

        def fire_in(i, u):
            pltpu.async_copy(tt_hbm.at[:, pl.ds(voff(i), vb)], src.at[u], isem[u])

        def wait_in(i, u):
            pltpu.make_async_copy(
                tt_hbm.at[:, pl.ds(voff(i), vb)], src.at[u], isem[u]
            ).wait()

        def fire_out(i, u):
            pltpu.async_copy(
                dst.at[u, :, pl.ds(0, d)],
                out_hbm.at[pl.ds(voff(i), vb)],
                osem[u],
            )

        def wait_out(i, u):
            pltpu.make_async_copy(
                dst.at[u, :, pl.ds(0, d)],
                out_hbm.at[pl.ds(voff(i), vb)],
                osem[u],
            ).wait()

        jvq = [lax.iota(jnp.int32, _L) + _L * q for q in range(vb // _L)]

        def transpose(u):
            s = src.at[u]
            t = dst.at[u]
            for c in range(0, d, 4):
                vals = [
                    [s[c + i, pl.ds(_L * q, _L)] for q in range(vb // _L)]
                    for i in range(4)
                ]
                cvs = [jnp.full((_L,), c + i, jnp.int32) for i in range(4)]
                for i in range(4):
                    for q in range(vb // _L):
                        plsc.store_scatter(t, [jvq[q], cvs[i]], vals[i][q])

        def step(i, u, wait_o, fire_i):
            wait_in(i, u)
            if wait_o:
                wait_out(i - 2, u)
            transpose(u)
            fire_out(i, u)
            if fire_i:
                fire_in(i + 2, u)

        fire_in(0, 0)
        fire_in(1, 1)
        for i in range(2):
            step(i, i, wait_o=False, fire_i=True)

        def body(g, carry):
            i = 2 * g
            step(i, 0, wait_o=True, fire_i=True)
            step(i + 1, 1, wait_o=True, fire_i=True)
            return carry

        lax.fori_loop(1, per_w // 2 - 1, body, 0)
        for u in range(2):
            i = per_w - 2 + u
            step(i, u, wait_o=True, fire_i=False)
        for u in range(2):
            wait_out(per_w - 2 + u, u)

    return k


def kernel(x, table):
    b, h = x.shape
    v, d = table.shape
    xt = jnp.transpose(x).astype(jnp.int32)
    table_rm = _make_table_transpose(v, d)(jnp.transpose(table))
    out_t = _make_kernel(b, h, v, d)(xt, table_rm)
    return jnp.transpose(out_t, (2, 0, 1))


# speedup vs baseline: 5.5608x; 5.5608x over previous
import functools

import jax
import jax.numpy as jnp
from jax import lax
from jax.experimental import pallas as pl
from jax.experimental.pallas import tpu as pltpu
from jax.experimental.pallas import tpu_sc as plsc

_L = 16


@functools.cache
def _make_kernel(b, h, v, d):
    info = plsc.get_sparse_core_info()
    nc, ns = info.num_cores, info.num_subcores
    nw = nc * ns
    bw = b // nw
    mesh = plsc.VectorSubcoreMesh(core_axis_name="c", subcore_axis_name="s")

    @functools.partial(
        pl.kernel,
        mesh=mesh,
        out_type=jax.ShapeDtypeStruct((h, d, b), jnp.float32),
        compiler_params=pltpu.CompilerParams(
            use_tc_tiling_on_sc=False, needs_layout_passes=False
        ),
        scratch_types=[
            pltpu.VMEM((h, bw), jnp.int32),
            pltpu.VMEM((4, bw, d), jnp.float32),
            pltpu.VMEM((2, d, bw + 1), jnp.float32),
        ]
        + [pltpu.SemaphoreType.DMA] * 6,
    )
    def k(xt_hbm, table_hbm, out_hbm, xv, rows, tst, g0, g1, g2, g3, s0, s1):
        gsem = (g0, g1, g2, g3)
        ssem = (s0, s1)
        wid = lax.axis_index("s") * nc + lax.axis_index("c")
        b0 = wid * bw
        pltpu.sync_copy(xt_hbm.at[:, pl.ds(b0, bw)], xv)

        def fire_gather(hh, buf):
            pltpu.async_copy(table_hbm.at[xv.at[hh]], rows.at[buf], gsem[buf])

        def wait_gather(hh, buf):
            pltpu.make_async_copy(
                table_hbm.at[xv.at[hh]], rows.at[buf], gsem[buf]
            ).wait()

        def fire_store(hh, buf):
            pltpu.async_copy(
                tst.at[buf, :, pl.ds(0, bw)],
                out_hbm.at[hh, :, pl.ds(b0, bw)],
                ssem[buf],
            )

        def wait_store(hh, buf):
            pltpu.make_async_copy(
                tst.at[buf, :, pl.ds(0, bw)],
                out_hbm.at[hh, :, pl.ds(b0, bw)],
                ssem[buf],
            ).wait()

        cqs = [lax.iota(jnp.int32, _L) + _L * q for q in range(d // _L)]

        def transpose(gbuf, tbuf):
            src = rows.at[gbuf]
            dst = tst.at[tbuf]
            nq = d // _L

            def grp(i, carry):
                j0 = i * 32
                for j in range(0, 32, 4):
                    vals = [
                        [src[j0 + j + u, pl.ds(_L * q, _L)] for q in range(nq)]
                        for u in range(4)
                    ]
                    jvs = [
                        jnp.full((_L,), j + u, jnp.int32) + j0 for u in range(4)
                    ]
                    for u in range(4):
                        for q in range(nq):
                            plsc.store_scatter(dst, [cqs[q], jvs[u]], vals[u][q])
                return carry

            lax.fori_loop(0, bw // 32, grp, 0)

        def step(hh, u, wait_st, fire_g):
            wait_gather(hh, u)
            if wait_st:
                wait_store(hh - 2, u % 2)
            transpose(u, u % 2)
            fire_store(hh, u % 2)
            if fire_g:
                fire_gather(hh + 3, (u + 3) % 4)

        for hh in range(3):
            fire_gather(hh, hh)
        for hh in range(4):
            step(hh, hh, wait_st=hh >= 2, fire_g=True)

        def body(gidx, carry):
            h0 = 4 + gidx * 4
            for u in range(4):
                step(h0 + u, u, wait_st=True, fire_g=True)
            return carry

        lax.fori_loop(0, (h - 8) // 4, body, 0)
        for hh in range(h - 4, h):
            step(hh, hh % 4, wait_st=True, fire_g=hh + 3 < h)
        for u in range(2):
            wait_store(h - 2 + u, u)

    return k


def kernel(x, table):
    b, h = x.shape
    v, d = table.shape
    xt = jnp.transpose(x).astype(jnp.int32)
    table_rm = table
    out_t = _make_kernel(b, h, v, d)(xt, table_rm)
    return jnp.transpose(out_t, (2, 0, 1))
